# baseline (speedup 1.0000x reference)
import functools

import jax
import jax.numpy as jnp
from jax import lax
from jax.experimental import pallas as pl
from jax.experimental.pallas import tpu as pltpu
from jax.experimental.pallas import tpu_sc as plsc

N_TOKENS = 131072
N_ACTORS = 65536
D_MODEL = 256
N_CHOICE = 64

TOK_BLK = 2048
ROW_BLK = 4096
GATHER_WIN = 512


def _project_body(x_ref, wt_ref, b_ref, out_ref):
    out_ref[...] = jnp.dot(
        x_ref[...].astype(jnp.bfloat16), wt_ref[...],
        preferred_element_type=jnp.float32,
    ) + b_ref[...]


def _project(x, wt, b2):
    return pl.pallas_call(
        _project_body,
        grid=(N_TOKENS // TOK_BLK,),
        in_specs=[
            pl.BlockSpec((TOK_BLK, D_MODEL), lambda i: (i, 0)),
            pl.BlockSpec((D_MODEL, N_CHOICE), lambda i: (0, 0)),
            pl.BlockSpec((1, N_CHOICE), lambda i: (0, 0)),
        ],
        out_specs=pl.BlockSpec((TOK_BLK, N_CHOICE), lambda i: (i, 0)),
        out_shape=jax.ShapeDtypeStruct((N_TOKENS, N_CHOICE), jnp.float32),
        compiler_params=pltpu.CompilerParams(
            dimension_semantics=("parallel",),
        ),
    )(x, wt, b2)


def _sc_gather(table, idx2):
    mesh = plsc.VectorSubcoreMesh(core_axis_name="c", subcore_axis_name="s")

    @functools.partial(
        pl.kernel,
        out_type=jax.ShapeDtypeStruct((N_ACTORS, N_CHOICE), jnp.float32),
        mesh=mesh,
        compiler_params=pltpu.CompilerParams(use_tc_tiling_on_sc=False),
    )
    def k(p_hbm, i_hbm, o_hbm):
        def body(i_vmem, o_vmem):
            pltpu.sync_copy(p_hbm.at[i_vmem.at[0]], o_vmem)

        pltpu.emit_pipeline(
            body,
            grid=(N_ACTORS // GATHER_WIN,),
            in_specs=[pl.BlockSpec((1, GATHER_WIN), lambda i: (0, i))],
            out_specs=[pl.BlockSpec((GATHER_WIN, N_CHOICE), lambda i: (i, 0))],
            core_axis_name=("c", "s"),
            dimension_semantics=(pltpu.PARALLEL,),
        )(i_hbm, o_hbm)

    return k(table, idx2)


def _head_body(g_ref, m_ref, a_ref, norm_ref, lp_ref, ent_ref):
    g = g_ref[...]
    legal = m_ref[...] != 0
    neg_inf = jnp.float32(-jnp.inf)
    logits = jnp.where(legal, g, neg_inf)
    mx = jnp.max(logits, axis=1, keepdims=True)
    ex = jnp.where(legal, jnp.exp(logits - mx), jnp.float32(0.0))
    s = jnp.sum(ex, axis=1, keepdims=True)
    lse = mx + jnp.log(s)
    norm = jnp.where(legal, logits - lse, neg_inf)
    norm_ref[...] = norm
    probs = ex / s
    plp = jnp.where(legal, norm * probs, jnp.float32(0.0))
    ent_ref[...] = -jnp.sum(plp, axis=1, keepdims=True)
    a = a_ref[...]
    col = lax.broadcasted_iota(jnp.int32, (g.shape[0], N_CHOICE), 1)
    sel = col == a
    lp_ref[...] = jnp.sum(jnp.where(sel, norm, jnp.float32(0.0)), axis=1,
                          keepdims=True)


def _head(g, mask, prev2):
    return pl.pallas_call(
        _head_body,
        grid=(N_ACTORS // ROW_BLK,),
        in_specs=[
            pl.BlockSpec((ROW_BLK, N_CHOICE), lambda i: (i, 0)),
            pl.BlockSpec((ROW_BLK, N_CHOICE), lambda i: (i, 0)),
            pl.BlockSpec((ROW_BLK, 1), lambda i: (i, 0)),
        ],
        out_specs=[
            pl.BlockSpec((ROW_BLK, N_CHOICE), lambda i: (i, 0)),
            pl.BlockSpec((ROW_BLK, 1), lambda i: (i, 0)),
            pl.BlockSpec((ROW_BLK, 1), lambda i: (i, 0)),
        ],
        out_shape=[
            jax.ShapeDtypeStruct((N_ACTORS, N_CHOICE), jnp.float32),
            jax.ShapeDtypeStruct((N_ACTORS, 1), jnp.float32),
            jax.ShapeDtypeStruct((N_ACTORS, 1), jnp.float32),
        ],
        compiler_params=pltpu.CompilerParams(
            dimension_semantics=("parallel",),
        ),
    )(g, mask, prev2)


def kernel(x_data, actors, mask, prev_actions, W, b):
    wt = W.T.astype(jnp.bfloat16)
    b2 = b.reshape(1, N_CHOICE)
    p = _project(x_data, wt, b2)
    idx2 = actors.astype(jnp.int32).reshape(1, N_ACTORS)
    g = _sc_gather(p, idx2)
    prev2 = prev_actions.reshape(N_ACTORS, 1)
    norm, lp, ent = _head(g, mask, prev2)
    return (prev_actions, lp[:, 0], ent[:, 0], norm)


# speedup vs baseline: 2.6565x; 2.6565x over previous
import functools

import jax
import jax.numpy as jnp
from jax import lax
from jax.experimental import pallas as pl
from jax.experimental.pallas import tpu as pltpu
from jax.experimental.pallas import tpu_sc as plsc

N_TOKENS = 131072
N_ACTORS = 65536
D_MODEL = 256
N_CHOICE = 64
C_PAD = 128

TOK_BLK = 8192
ROW_BLK = 8192
GATHER_WIN = 256


def _project_body(x_ref, wt_ref, b_ref, out_ref):
    r = jnp.dot(
        x_ref[...].astype(jnp.bfloat16), wt_ref[...],
        preferred_element_type=jnp.float32,
    ) + b_ref[...]
    out_ref[...] = jnp.concatenate(
        [r[:TOK_BLK // 2], r[TOK_BLK // 2:]], axis=1)


def _project(x, wt, b2):
    return pl.pallas_call(
        _project_body,
        grid=(N_TOKENS // TOK_BLK,),
        in_specs=[
            pl.BlockSpec((TOK_BLK, D_MODEL), lambda i: (i, 0)),
            pl.BlockSpec((D_MODEL, N_CHOICE), lambda i: (0, 0)),
            pl.BlockSpec((1, N_CHOICE), lambda i: (0, 0)),
        ],
        out_specs=pl.BlockSpec((TOK_BLK // 2, C_PAD), lambda i: (i, 0)),
        out_shape=jax.ShapeDtypeStruct((N_TOKENS // 2, C_PAD), jnp.float32),
        compiler_params=pltpu.CompilerParams(
            dimension_semantics=("parallel",),
        ),
    )(x, wt, b2)


def _sc_gather(table, idx2):
    mesh = plsc.VectorSubcoreMesh(core_axis_name="c", subcore_axis_name="s")

    @functools.partial(
        pl.kernel,
        out_type=jax.ShapeDtypeStruct((N_ACTORS, C_PAD), jnp.float32),
        mesh=mesh,
    )
    def k(p_hbm, i_hbm, o_hbm):
        def body(i_vmem, o_vmem):
            pltpu.sync_copy(p_hbm.at[i_vmem.at[0]], o_vmem)

        pltpu.emit_pipeline(
            body,
            grid=(N_ACTORS // GATHER_WIN,),
            in_specs=[pl.BlockSpec((1, GATHER_WIN), lambda i: (0, i))],
            out_specs=[pl.BlockSpec((GATHER_WIN, C_PAD), lambda i: (i, 0))],
            core_axis_name=("c", "s"),
            dimension_semantics=(pltpu.PARALLEL,),
        )(i_hbm, o_hbm)

    return k(table, idx2)


def _head_body(g_ref, m_ref, a_ref, h_ref, norm_ref, lp_ref, ent_ref):
    gfull = jnp.transpose(g_ref[...])
    gt = jnp.where((h_ref[...] != 0)[None, :],
                   gfull[N_CHOICE:], gfull[:N_CHOICE])
    legal = m_ref[...] != 0
    neg_inf = jnp.float32(-jnp.inf)
    logits = jnp.where(legal, gt, neg_inf)
    mx = jnp.max(logits, axis=0, keepdims=True)
    ex = jnp.exp(logits - mx)
    s = jnp.sum(ex, axis=0, keepdims=True)
    lse = mx + jnp.log(s)
    norm = logits - lse
    norm_ref[...] = norm
    probs = ex / s
    plp = jnp.where(legal, norm * probs, jnp.float32(0.0))
    ent_ref[...] = -jnp.sum(plp, axis=0)
    a = a_ref[...]
    row = lax.broadcasted_iota(jnp.int32, (N_CHOICE, ROW_BLK), 0)
    sel = row == a[None, :]
    lp_ref[...] = jnp.sum(jnp.where(sel, norm, jnp.float32(0.0)), axis=0)


def _head(g, mask_t, prev, half):
    return pl.pallas_call(
        _head_body,
        grid=(N_ACTORS // ROW_BLK,),
        in_specs=[
            pl.BlockSpec((ROW_BLK, C_PAD), lambda i: (i, 0)),
            pl.BlockSpec((N_CHOICE, ROW_BLK), lambda i: (0, i)),
            pl.BlockSpec((ROW_BLK,), lambda i: (i,)),
            pl.BlockSpec((ROW_BLK,), lambda i: (i,)),
        ],
        out_specs=[
            pl.BlockSpec((N_CHOICE, ROW_BLK), lambda i: (0, i)),
            pl.BlockSpec((ROW_BLK,), lambda i: (i,)),
            pl.BlockSpec((ROW_BLK,), lambda i: (i,)),
        ],
        out_shape=[
            jax.ShapeDtypeStruct((N_CHOICE, N_ACTORS), jnp.float32),
            jax.ShapeDtypeStruct((N_ACTORS,), jnp.float32),
            jax.ShapeDtypeStruct((N_ACTORS,), jnp.float32),
        ],
        compiler_params=pltpu.CompilerParams(
            dimension_semantics=("parallel",),
        ),
    )(g, mask_t, prev, half)


def kernel(x_data, actors, mask, prev_actions, W, b):
    wt = W.T.astype(jnp.bfloat16)
    b2 = b.reshape(1, N_CHOICE)
    p = _project(x_data, wt, b2)
    t = actors.astype(jnp.int32)
    hb = TOK_BLK // 2
    rows = (t // TOK_BLK) * hb + (t % hb)
    half = (t % TOK_BLK) // hb
    idx2 = rows.reshape(1, N_ACTORS)
    g = _sc_gather(p, idx2)
    norm_t, lp, ent = _head(g, mask.T, prev_actions, half)
    return (prev_actions, lp, ent, norm_t.T)
